# baseline (speedup 1.0000x reference)
import functools

import jax
import jax.numpy as jnp
from jax.experimental import pallas as pl
from jax.experimental.pallas import tpu as pltpu
from jax.experimental.pallas import tpu_sc as plsc

PERCENT_ON = 0.1


def _make_tc_body(BM, BN, NB, K):
    def _body(x_ref, w_ref, b_ref, o_ref, t_ref, s_ref):
        n = pl.program_id(1)
        h = jax.lax.dot_general(
            x_ref[...], w_ref[...], (((1,), (1,)), ((), ())),
            preferred_element_type=jnp.float32)
        o_ref[:, pl.ds(n * BN, BN)] = h + b_ref[...]

        @pl.when(n == NB - 1)
        def _select():
            z = o_ref[...]
            rmin = jnp.min(z, axis=1, keepdims=True)
            rmax = jnp.max(z, axis=1, keepdims=True)
            hn = (z - rmin) / (rmax - rmin)
            u = jax.lax.bitcast_convert_type(hn, jnp.int32)

            def step(i, t):
                cand = t | (jnp.int32(1) << (29 - i))
                cnt = jnp.sum((u >= cand).astype(jnp.int32), axis=1,
                              keepdims=True)
                return jnp.where(cnt >= K, cand, t)

            t = jax.lax.fori_loop(0, 30, step,
                                  jnp.zeros((BM, 1), jnp.int32))
            f = jnp.where(u >= t, hn, 0.0)
            ssq = jnp.sum(f * f, axis=1, keepdims=True)
            o_ref[...] = hn
            tf = jax.lax.bitcast_convert_type(t, jnp.float32)
            t_ref[...] = jnp.broadcast_to(tf, tf.shape[:1] + (16,))
            sv = 1.0 / jnp.maximum(jnp.sqrt(ssq), 1e-12)
            s_ref[...] = jnp.broadcast_to(sv, sv.shape[:1] + (16,))

    return _body


def _tc_stage(x, W, b, K):
    B, D = x.shape
    N = W.shape[0]
    BM = min(256, B)
    BN = min(1024, N)
    NB = N // BN
    grid = (B // BM, NB)
    return pl.pallas_call(
        _make_tc_body(BM, BN, NB, K),
        grid=grid,
        in_specs=[
            pl.BlockSpec((BM, D), lambda i, n: (i, 0)),
            pl.BlockSpec((BN, D), lambda i, n: (n, 0)),
            pl.BlockSpec((1, BN), lambda i, n: (0, n)),
        ],
        out_specs=[
            pl.BlockSpec((BM, N), lambda i, n: (i, 0)),
            pl.BlockSpec((BM, 16), lambda i, n: (i, 0)),
            pl.BlockSpec((BM, 16), lambda i, n: (i, 0)),
        ],
        out_shape=[
            jax.ShapeDtypeStruct((B, N), jnp.float32),
            jax.ShapeDtypeStruct((B, 16), jnp.float32),
            jax.ShapeDtypeStruct((B, 16), jnp.float32),
        ],
        compiler_params=pltpu.CompilerParams(
            dimension_semantics=("parallel", "arbitrary"),
        ),
    )(x, W, b.reshape(1, N))


def _sc_stage(hn, tb, sb):
    B, N = hn.shape
    NC, NS, L = 2, 16, 16
    NW = NC * NS
    RPW = B // NW
    CH = 16
    NCH = RPW // CH
    U = 8
    mesh = plsc.VectorSubcoreMesh(core_axis_name="c", subcore_axis_name="s")

    @functools.partial(
        pl.kernel, mesh=mesh,
        out_type=jax.ShapeDtypeStruct((B, N), jnp.float32),
        scratch_types=[
            pltpu.VMEM((CH, N), jnp.float32),
            pltpu.VMEM((L,), jnp.float32),
            pltpu.VMEM((L,), jnp.float32),
        ],
    )
    def k(hn_hbm, t_hbm, s_hbm, out_hbm, buf, tv, sv):
        wid = jax.lax.axis_index("s") * NC + jax.lax.axis_index("c")
        base = wid * RPW

        def chunk(ci, carry):
            r0 = base + ci * CH
            pltpu.sync_copy(hn_hbm.at[pl.ds(r0, CH)], buf)
            for r in range(CH):
                pltpu.sync_copy(t_hbm.at[r0 + r], tv)
                pltpu.sync_copy(s_hbm.at[r0 + r], sv)
                tr = tv[...]
                sr = sv[...]

                def cols(c, inner, r=r, tr=tr, sr=sr):
                    for k8 in range(U):
                        sl = pl.ds((c * U + k8) * L, L)
                        v = buf[r, sl]
                        buf[r, sl] = jnp.where(v >= tr, v * sr, 0.0)
                    return inner

                jax.lax.fori_loop(0, N // (L * U), cols, 0)
            pltpu.sync_copy(buf, out_hbm.at[pl.ds(r0, CH)])
            return carry

        jax.lax.fori_loop(0, NCH, chunk, 0)

    return k(hn, tb, sb)


def kernel(x, W, b):
    N = W.shape[0]
    K = int(round(N * PERCENT_ON))
    hn, tb, sb = _tc_stage(x, W, b, K)
    return _sc_stage(hn, tb, sb)


# speedup vs baseline: 1.7687x; 1.7687x over previous
import jax
import jax.numpy as jnp
from jax.experimental import pallas as pl
from jax.experimental.pallas import tpu as pltpu

PERCENT_ON = 0.1


def _make_body(BM, BN, NB, K):
    def _body(x_ref, w_ref, b_ref, o_ref):
        n = pl.program_id(1)
        h = jax.lax.dot_general(
            x_ref[...], w_ref[...], (((1,), (1,)), ((), ())),
            preferred_element_type=jnp.float32)
        o_ref[:, pl.ds(n * BN, BN)] = h + b_ref[...]

        @pl.when(n == NB - 1)
        def _select():
            z = o_ref[...]
            rmin = jnp.min(z, axis=1, keepdims=True)
            rmax = jnp.max(z, axis=1, keepdims=True)
            hn = (z - rmin) / (rmax - rmin)
            u = jax.lax.bitcast_convert_type(hn, jnp.int32)

            def step(i, t):
                cand = t | (jnp.int32(1) << (29 - i))
                cnt = jnp.sum((u >= cand).astype(jnp.int32), axis=1,
                              keepdims=True)
                return jnp.where(cnt >= K, cand, t)

            t = jax.lax.fori_loop(0, 30, step,
                                  jnp.zeros((BM, 1), jnp.int32))
            f = jnp.where(u >= t, hn, 0.0)
            ssq = jnp.sum(f * f, axis=1, keepdims=True)
            o_ref[...] = f / jnp.maximum(jnp.sqrt(ssq), 1e-12)

    return _body


def kernel(x, W, b):
    B, D = x.shape
    N = W.shape[0]
    K = int(round(N * PERCENT_ON))
    BM = min(512, B)
    BN = min(1024, N)
    NB = N // BN
    grid = (B // BM, NB)
    return pl.pallas_call(
        _make_body(BM, BN, NB, K),
        grid=grid,
        in_specs=[
            pl.BlockSpec((BM, D), lambda i, n: (i, 0)),
            pl.BlockSpec((BN, D), lambda i, n: (n, 0)),
            pl.BlockSpec((1, BN), lambda i, n: (0, n)),
        ],
        out_specs=pl.BlockSpec((BM, N), lambda i, n: (i, 0)),
        out_shape=jax.ShapeDtypeStruct((B, N), jnp.float32),
        compiler_params=pltpu.CompilerParams(
            dimension_semantics=("parallel", "arbitrary"),
        ),
    )(x, W, b.reshape(1, N))
